# baseline (speedup 1.0000x reference)
import functools

import jax
import jax.numpy as jnp
from jax.experimental import pallas as pl
from jax.experimental.pallas import tpu as pltpu

_Q = 8
_K = 1024
_D = 256
_BLK = 1024
_ROWS = 8 * 1024


def _rvq_block_kernel(x_ref, cb_ref, cbt_ref, c0_ref, c1_ref, c2_ref,
                      out_ref, idx_ref, loss_ref, qs_ref):
    r = x_ref[...]
    qout = jnp.zeros_like(r)
    lane_iota = jax.lax.broadcasted_iota(jnp.int32, (_BLK, _K), 1)
    idx_cols = []
    loss_cols = []
    for q in range(_Q):
        e = cb_ref[q]
        et = cbt_ref[q]
        rn_r = jnp.sum(r * r, axis=1, keepdims=True)
        rn_e = jnp.sum(e * e, axis=1)
        mm = jax.lax.dot(2.0 * r, et)
        dist = (rn_r - mm) + rn_e[None, :]
        idx = jnp.argmin(dist, axis=1, keepdims=True).astype(jnp.int32)
        onehot = (lane_iota == idx).astype(jnp.float32)
        qs_ref[...] = jax.lax.dot(onehot, c0_ref[q])
        qs_ref[...] = qs_ref[...] + jax.lax.dot(onehot, c1_ref[q])
        qs_ref[...] = qs_ref[...] + jax.lax.dot(onehot, c2_ref[q])
        quant = qs_ref[...]
        diff = quant - r
        lsum = jnp.sum((diff * diff).reshape(1, -1), axis=1, keepdims=True)
        qst = r + (quant - r)
        r = r - qst
        qout = qout + qst
        idx_cols.append(idx)
        loss_cols.append(lsum)
    out_ref[...] = qout
    idx_ref[...] = jnp.concatenate(idx_cols, axis=1)
    loss_ref[...] = jnp.concatenate(loss_cols, axis=1).reshape(1, 1, _Q)


@jax.jit
def kernel(x, codebooks):
    b, n, d = x.shape
    flat = x.reshape(-1, d)
    cbt = codebooks.transpose(0, 2, 1)
    c0 = jax.lax.optimization_barrier(
        codebooks.astype(jnp.bfloat16)).astype(jnp.float32)
    r1 = codebooks - c0
    c1 = jax.lax.optimization_barrier(
        r1.astype(jnp.bfloat16)).astype(jnp.float32)
    c2 = jax.lax.optimization_barrier(
        (r1 - c1).astype(jnp.bfloat16)).astype(jnp.float32)
    nblk = _ROWS // _BLK
    out, idx, loss_part = pl.pallas_call(
        _rvq_block_kernel,
        grid=(nblk,),
        in_specs=[
            pl.BlockSpec((_BLK, _D), lambda i: (i, 0)),
            pl.BlockSpec((_Q, _K, _D), lambda i: (0, 0, 0)),
            pl.BlockSpec((_Q, _D, _K), lambda i: (0, 0, 0)),
            pl.BlockSpec((_Q, _K, _D), lambda i: (0, 0, 0)),
            pl.BlockSpec((_Q, _K, _D), lambda i: (0, 0, 0)),
            pl.BlockSpec((_Q, _K, _D), lambda i: (0, 0, 0)),
        ],
        out_specs=[
            pl.BlockSpec((_BLK, _D), lambda i: (i, 0)),
            pl.BlockSpec((_BLK, _Q), lambda i: (i, 0)),
            pl.BlockSpec((1, 1, _Q), lambda i: (i, 0, 0)),
        ],
        out_shape=[
            jax.ShapeDtypeStruct((_ROWS, _D), jnp.float32),
            jax.ShapeDtypeStruct((_ROWS, _Q), jnp.int32),
            jax.ShapeDtypeStruct((nblk, 1, _Q), jnp.float32),
        ],
        scratch_shapes=[pltpu.VMEM((_BLK, _D), jnp.float32)],
        compiler_params=pltpu.CompilerParams(
            dimension_semantics=("arbitrary",),
        ),
    )(flat, codebooks, cbt, c0, c1, c2)
    quantized_out = out.reshape(b, n, d)
    all_indices = idx.reshape(b, n, _Q)
    all_losses = (jnp.sum(loss_part, axis=(0, 1)) / (b * n * d)).astype(
        jnp.float32)
    return quantized_out, all_indices, all_losses


# speedup vs baseline: 1.2060x; 1.2060x over previous
import functools

import jax
import jax.numpy as jnp
from jax.experimental import pallas as pl
from jax.experimental.pallas import tpu as pltpu

_Q = 8
_K = 1024
_D = 256
_BLK = 1024
_ROWS = 8 * 1024


def _rvq_block_kernel(x_ref, cb_ref, cbt_ref, c0_ref, c1_ref, c2_ref,
                      out_ref, idx_ref, loss_ref, qs_ref):
    r = x_ref[...]
    qout = jnp.zeros_like(r)
    lane_iota = jax.lax.broadcasted_iota(jnp.int32, (_BLK, _K), 1)
    idx_cols = []
    loss_cols = []
    for q in range(_Q):
        e = cb_ref[q]
        et = cbt_ref[q]
        rn_r = jnp.sum(r * r, axis=1, keepdims=True)
        rn_e = jnp.sum(e * e, axis=1)
        mm = jax.lax.dot(2.0 * r, et)
        dist = (rn_r - mm) + rn_e[None, :]
        idx = jnp.argmin(dist, axis=1, keepdims=True).astype(jnp.int32)
        onehot = (lane_iota == idx).astype(jnp.bfloat16)
        qs_ref[...] = jax.lax.dot(onehot, c0_ref[q],
                                  preferred_element_type=jnp.float32)
        qs_ref[...] = qs_ref[...] + jax.lax.dot(
            onehot, c1_ref[q], preferred_element_type=jnp.float32)
        qs_ref[...] = qs_ref[...] + jax.lax.dot(
            onehot, c2_ref[q], preferred_element_type=jnp.float32)
        quant = qs_ref[...]
        diff = quant - r
        lsum = jnp.sum((diff * diff).reshape(1, -1), axis=1, keepdims=True)
        qst = r + (quant - r)
        r = r - qst
        qout = qout + qst
        idx_cols.append(idx)
        loss_cols.append(lsum)
    out_ref[...] = qout
    idx_ref[...] = jnp.concatenate(idx_cols, axis=1)
    loss_ref[...] = jnp.concatenate(loss_cols, axis=1).reshape(1, 1, _Q)


@jax.jit
def kernel(x, codebooks):
    b, n, d = x.shape
    flat = x.reshape(-1, d)
    cbt = codebooks.transpose(0, 2, 1)
    c0 = jax.lax.optimization_barrier(codebooks.astype(jnp.bfloat16))
    r1 = codebooks - c0.astype(jnp.float32)
    c1 = jax.lax.optimization_barrier(r1.astype(jnp.bfloat16))
    c2 = jax.lax.optimization_barrier(
        (r1 - c1.astype(jnp.float32)).astype(jnp.bfloat16))
    nblk = _ROWS // _BLK
    out, idx, loss_part = pl.pallas_call(
        _rvq_block_kernel,
        grid=(nblk,),
        in_specs=[
            pl.BlockSpec((_BLK, _D), lambda i: (i, 0)),
            pl.BlockSpec((_Q, _K, _D), lambda i: (0, 0, 0)),
            pl.BlockSpec((_Q, _D, _K), lambda i: (0, 0, 0)),
            pl.BlockSpec((_Q, _K, _D), lambda i: (0, 0, 0)),
            pl.BlockSpec((_Q, _K, _D), lambda i: (0, 0, 0)),
            pl.BlockSpec((_Q, _K, _D), lambda i: (0, 0, 0)),
        ],
        out_specs=[
            pl.BlockSpec((_BLK, _D), lambda i: (i, 0)),
            pl.BlockSpec((_BLK, _Q), lambda i: (i, 0)),
            pl.BlockSpec((1, 1, _Q), lambda i: (i, 0, 0)),
        ],
        out_shape=[
            jax.ShapeDtypeStruct((_ROWS, _D), jnp.float32),
            jax.ShapeDtypeStruct((_ROWS, _Q), jnp.int32),
            jax.ShapeDtypeStruct((nblk, 1, _Q), jnp.float32),
        ],
        scratch_shapes=[pltpu.VMEM((_BLK, _D), jnp.float32)],
        compiler_params=pltpu.CompilerParams(
            dimension_semantics=("arbitrary",),
        ),
    )(flat, codebooks, cbt, c0, c1, c2)
    quantized_out = out.reshape(b, n, d)
    all_indices = idx.reshape(b, n, _Q)
    all_losses = (jnp.sum(loss_part, axis=(0, 1)) / (b * n * d)).astype(
        jnp.float32)
    return quantized_out, all_indices, all_losses


# speedup vs baseline: 1.6426x; 1.3620x over previous
import functools

import jax
import jax.numpy as jnp
from jax.experimental import pallas as pl
from jax.experimental.pallas import tpu as pltpu

_Q = 8
_K = 1024
_D = 256
_BLK = 1024
_ROWS = 8 * 1024


_H = 2
_S = _BLK // _H


def _rvq_block_kernel(x_ref, cb_ref, cbt_ref, c0_ref, c1_ref, c2_ref,
                      out_ref, idx_ref, loss_ref, *qs_refs):
    lane_iota = jax.lax.broadcasted_iota(jnp.int32, (_S, _K), 1)
    rs = [x_ref[pl.ds(h * _S, _S), :] for h in range(_H)]
    qouts = [jnp.zeros_like(rs[h]) for h in range(_H)]
    idx_cols = [[] for _ in range(_H)]
    loss_cols = []
    for q in range(_Q):
        e = cb_ref[q]
        et = cbt_ref[q]
        rn_e = jnp.sum(e * e, axis=1)
        lsums = []
        for h in range(_H):
            r = rs[h]
            qs_ref = qs_refs[h]
            rn_r = jnp.sum(r * r, axis=1, keepdims=True)
            mm = jax.lax.dot(2.0 * r, et)
            dist = (rn_r - mm) + rn_e[None, :]
            idx = jnp.argmin(dist, axis=1, keepdims=True).astype(jnp.int32)
            onehot = (lane_iota == idx).astype(jnp.bfloat16)
            qs_ref[...] = jax.lax.dot(onehot, c0_ref[q],
                                      preferred_element_type=jnp.float32)
            qs_ref[...] = qs_ref[...] + jax.lax.dot(
                onehot, c1_ref[q], preferred_element_type=jnp.float32)
            qs_ref[...] = qs_ref[...] + jax.lax.dot(
                onehot, c2_ref[q], preferred_element_type=jnp.float32)
            quant = qs_ref[...]
            diff = quant - r
            lsums.append(jnp.sum((diff * diff).reshape(1, -1), axis=1,
                                 keepdims=True))
            qst = r + (quant - r)
            rs[h] = r - qst
            qouts[h] = qouts[h] + qst
            idx_cols[h].append(idx)
        lsum = lsums[0]
        for t in lsums[1:]:
            lsum = lsum + t
        loss_cols.append(lsum)
    for h in range(_H):
        out_ref[pl.ds(h * _S, _S), :] = qouts[h]
        idx_ref[pl.ds(h * _S, _S), :] = jnp.concatenate(idx_cols[h], axis=1)
    loss_ref[...] = jnp.concatenate(loss_cols, axis=1).reshape(1, 1, _Q)


@jax.jit
def kernel(x, codebooks):
    b, n, d = x.shape
    flat = x.reshape(-1, d)
    cbt = codebooks.transpose(0, 2, 1)
    c0 = jax.lax.optimization_barrier(codebooks.astype(jnp.bfloat16))
    r1 = codebooks - c0.astype(jnp.float32)
    c1 = jax.lax.optimization_barrier(r1.astype(jnp.bfloat16))
    c2 = jax.lax.optimization_barrier(
        (r1 - c1.astype(jnp.float32)).astype(jnp.bfloat16))
    nblk = _ROWS // _BLK
    out, idx, loss_part = pl.pallas_call(
        _rvq_block_kernel,
        grid=(nblk,),
        in_specs=[
            pl.BlockSpec((_BLK, _D), lambda i: (i, 0)),
            pl.BlockSpec((_Q, _K, _D), lambda i: (0, 0, 0)),
            pl.BlockSpec((_Q, _D, _K), lambda i: (0, 0, 0)),
            pl.BlockSpec((_Q, _K, _D), lambda i: (0, 0, 0)),
            pl.BlockSpec((_Q, _K, _D), lambda i: (0, 0, 0)),
            pl.BlockSpec((_Q, _K, _D), lambda i: (0, 0, 0)),
        ],
        out_specs=[
            pl.BlockSpec((_BLK, _D), lambda i: (i, 0)),
            pl.BlockSpec((_BLK, _Q), lambda i: (i, 0)),
            pl.BlockSpec((1, 1, _Q), lambda i: (i, 0, 0)),
        ],
        out_shape=[
            jax.ShapeDtypeStruct((_ROWS, _D), jnp.float32),
            jax.ShapeDtypeStruct((_ROWS, _Q), jnp.int32),
            jax.ShapeDtypeStruct((nblk, 1, _Q), jnp.float32),
        ],
        scratch_shapes=[pltpu.VMEM((_S, _D), jnp.float32)
                        for _ in range(_H)],
        compiler_params=pltpu.CompilerParams(
            dimension_semantics=("arbitrary",),
        ),
    )(flat, codebooks, cbt, c0, c1, c2)
    quantized_out = out.reshape(b, n, d)
    all_indices = idx.reshape(b, n, _Q)
    all_losses = (jnp.sum(loss_part, axis=(0, 1)) / (b * n * d)).astype(
        jnp.float32)
    return quantized_out, all_indices, all_losses
